# Initial kernel scaffold; baseline (speedup 1.0000x reference)
import jax
import jax.numpy as jnp
from jax.experimental import pallas as pl


def kernel(x, edge_index, Wxz, bxz, Whz, bhz, Wxr, bxr, Whr, bhr, Wxh, bxh, Whh, bhh, W_lin, b_lin):
    raise NotImplementedError("write your pallas kernel here")


# speedup vs baseline: 1.1753x; 1.1753x over previous
import functools

import jax
import jax.numpy as jnp
from jax.experimental import pallas as pl

_NUM_NODES_PER_GRAPH = 82
_HID = 30
_N = 38950
_ROW_BLOCK = 3968


def _gate_kernel(x_ref, wz_ref, wh_ref, bz_ref, bh_ref, o_ref):
    xb = x_ref[...]
    a = jnp.dot(xb, wz_ref[...], preferred_element_type=jnp.float32) + bz_ref[...]
    c = jnp.dot(xb, wh_ref[...], preferred_element_type=jnp.float32) + bh_ref[...]
    h = (1.0 - jax.nn.sigmoid(a)) * jnp.tanh(c)
    o_ref[...] = jnp.maximum(h, 0.0)


def _head_kernel(h_ref, w_ref, b_ref, o_ref):
    o_ref[...] = (
        jnp.dot(h_ref[...], w_ref[...], preferred_element_type=jnp.float32)
        + b_ref[...]
    )


@jax.jit
def kernel(x, edge_index, Wxz, bxz, Whz, bhz, Wxr, bxr, Whr, bhr, Wxh, bxh, Whh, bhh, W_lin, b_lin):
    n, d = x.shape
    hid = Wxz.shape[1]
    bz = (bxz + bhz).reshape(1, hid)
    bh = (bxh + bhh).reshape(1, hid)

    grid = pl.cdiv(n, _ROW_BLOCK)
    h = pl.pallas_call(
        _gate_kernel,
        grid=(grid,),
        in_specs=[
            pl.BlockSpec((_ROW_BLOCK, d), lambda i: (i, 0)),
            pl.BlockSpec((d, hid), lambda i: (0, 0)),
            pl.BlockSpec((d, hid), lambda i: (0, 0)),
            pl.BlockSpec((1, hid), lambda i: (0, 0)),
            pl.BlockSpec((1, hid), lambda i: (0, 0)),
        ],
        out_specs=pl.BlockSpec((_ROW_BLOCK, hid), lambda i: (i, 0)),
        out_shape=jax.ShapeDtypeStruct((n, hid), jnp.float32),
    )(x, Wxz, Wxh, bz, bh)

    feat = hid * _NUM_NODES_PER_GRAPH
    g = n // _NUM_NODES_PER_GRAPH
    hf = h.reshape(g, feat)
    w2 = W_lin.T
    out_dim = w2.shape[1]
    out = pl.pallas_call(
        _head_kernel,
        grid=(1,),
        in_specs=[
            pl.BlockSpec((g, feat), lambda i: (0, 0)),
            pl.BlockSpec((feat, out_dim), lambda i: (0, 0)),
            pl.BlockSpec((1, out_dim), lambda i: (0, 0)),
        ],
        out_specs=pl.BlockSpec((g, out_dim), lambda i: (0, 0)),
        out_shape=jax.ShapeDtypeStruct((g, out_dim), jnp.float32),
    )(hf, w2, b_lin.reshape(1, out_dim))
    return out
